# baseline (speedup 1.0000x reference)
import functools

import jax
import jax.numpy as jnp
from jax import lax
from jax.experimental import pallas as pl
from jax.experimental.pallas import tpu as pltpu
from jax.experimental.pallas import tpu_sc as plsc

VOCAB_SIZE = 100000
EMB_DIM = 100
EMB_PAD = 128
BATCH_SIZE = 1024
VT = 2048
NV = (VOCAB_SIZE + VT - 1) // VT


def _sc_gather(table, idx_flat):
    nc, ns = 2, 16
    nw = nc * ns
    n_idx = idx_flat.shape[0]
    per_w = n_idx // nw
    mesh = plsc.VectorSubcoreMesh(core_axis_name="c", subcore_axis_name="s",
                                  num_cores=nc, num_subcores=ns)

    @functools.partial(
        pl.kernel,
        mesh=mesh,
        out_type=jax.ShapeDtypeStruct((n_idx, EMB_PAD), jnp.float32),
        scratch_types=[
            pltpu.VMEM((per_w,), jnp.int32),
            pltpu.VMEM((per_w, EMB_PAD), jnp.float32),
            pltpu.SemaphoreType.DMA,
        ],
    )
    def gather_kernel(table_hbm, idx_hbm, out_hbm, idx_v, rows_v, sem):
        wid = lax.axis_index("s") * nc + lax.axis_index("c")
        base = wid * per_w
        pltpu.sync_copy(idx_hbm.at[pl.ds(base, per_w)], idx_v)
        pltpu.async_copy(table_hbm.at[idx_v], rows_v, sem).wait()
        pltpu.sync_copy(rows_v, out_hbm.at[pl.ds(base, per_w)])

    return gather_kernel(table, idx_flat)


def _logits_tile(rows_ref, w_ref, b_ref):
    t1 = rows_ref[0:BATCH_SIZE, 0:EMB_DIM]
    t2 = rows_ref[BATCH_SIZE:2 * BATCH_SIZE, 0:EMB_DIM]
    z = jnp.dot(t1, w_ref[0:EMB_DIM, :], preferred_element_type=jnp.float32)
    z = z + jnp.dot(t2, w_ref[EMB_DIM:2 * EMB_DIM, :],
                    preferred_element_type=jnp.float32)
    return z + b_ref[...]


def _pass1_body(rows_ref, w_ref, b_ref, m_ref, l_ref):
    j = pl.program_id(0)

    @pl.when(j == 0)
    def _():
        m_ref[...] = jnp.zeros_like(m_ref)
        l_ref[...] = jnp.zeros_like(l_ref)

    r = jnp.maximum(_logits_tile(rows_ref, w_ref, b_ref), 0.0)
    col = j * VT + lax.broadcasted_iota(jnp.int32, r.shape, 1)
    valid = col < VOCAB_SIZE
    r = jnp.where(valid, r, 0.0)
    tile_max = jnp.max(r, axis=1, keepdims=True)
    m_old = m_ref[...]
    m_new = jnp.maximum(m_old, tile_max)
    e = jnp.where(valid, jnp.exp(r - m_new), 0.0)
    l_ref[...] = l_ref[...] * jnp.exp(m_old - m_new) + jnp.sum(
        e, axis=1, keepdims=True)
    m_ref[...] = m_new


def _pass2_body(rows_ref, w_ref, b_ref, m_ref, l_ref, out_ref):
    r = jnp.maximum(_logits_tile(rows_ref, w_ref, b_ref), 0.0)
    out_ref[...] = jnp.exp(r - m_ref[...]) * (1.0 / l_ref[...])


def kernel(inputs, E, W, b):
    table = jnp.pad(E, ((0, 0), (0, EMB_PAD - EMB_DIM)))
    idx = jnp.concatenate([inputs[:, 0], inputs[:, 1]]).astype(jnp.int32)
    rows = _sc_gather(table, idx)
    b2 = b.reshape(1, VOCAB_SIZE)

    m, l = pl.pallas_call(
        _pass1_body,
        grid=(NV,),
        in_specs=[
            pl.BlockSpec((2 * BATCH_SIZE, EMB_PAD), lambda j: (0, 0)),
            pl.BlockSpec((2 * EMB_DIM, VT), lambda j: (0, j)),
            pl.BlockSpec((1, VT), lambda j: (0, j)),
        ],
        out_specs=[
            pl.BlockSpec((BATCH_SIZE, 1), lambda j: (0, 0)),
            pl.BlockSpec((BATCH_SIZE, 1), lambda j: (0, 0)),
        ],
        out_shape=[
            jax.ShapeDtypeStruct((BATCH_SIZE, 1), jnp.float32),
            jax.ShapeDtypeStruct((BATCH_SIZE, 1), jnp.float32),
        ],
        compiler_params=pltpu.CompilerParams(
            dimension_semantics=("arbitrary",)),
    )(rows, W, b2)

    out = pl.pallas_call(
        _pass2_body,
        grid=(NV,),
        in_specs=[
            pl.BlockSpec((2 * BATCH_SIZE, EMB_PAD), lambda j: (0, 0)),
            pl.BlockSpec((2 * EMB_DIM, VT), lambda j: (0, j)),
            pl.BlockSpec((1, VT), lambda j: (0, j)),
            pl.BlockSpec((BATCH_SIZE, 1), lambda j: (0, 0)),
            pl.BlockSpec((BATCH_SIZE, 1), lambda j: (0, 0)),
        ],
        out_specs=pl.BlockSpec((BATCH_SIZE, VT), lambda j: (0, j)),
        out_shape=jax.ShapeDtypeStruct((BATCH_SIZE, VOCAB_SIZE), jnp.float32),
        compiler_params=pltpu.CompilerParams(
            dimension_semantics=("arbitrary",)),
    )(rows, W, b2, m, l)
    return out


# speedup vs baseline: 1.2739x; 1.2739x over previous
import functools

import jax
import jax.numpy as jnp
from jax import lax
from jax.experimental import pallas as pl
from jax.experimental.pallas import tpu as pltpu
from jax.experimental.pallas import tpu_sc as plsc

VOCAB_SIZE = 100000
EMB_DIM = 100
EMB_PAD = 128
BATCH_SIZE = 1024
VT = 2048
NV = (VOCAB_SIZE + VT - 1) // VT
PAD_ROWS = 10000


def _pad_body(e_ref, out_ref):
    out_ref[:, 0:EMB_DIM] = e_ref[...]
    out_ref[:, EMB_DIM:EMB_PAD] = jnp.zeros(
        (PAD_ROWS, EMB_PAD - EMB_DIM), jnp.float32)


def _pad_table(E):
    return pl.pallas_call(
        _pad_body,
        grid=(VOCAB_SIZE // PAD_ROWS,),
        in_specs=[pl.BlockSpec((PAD_ROWS, EMB_DIM), lambda i: (i, 0))],
        out_specs=pl.BlockSpec((PAD_ROWS, EMB_PAD), lambda i: (i, 0)),
        out_shape=jax.ShapeDtypeStruct((VOCAB_SIZE, EMB_PAD), jnp.float32),
        compiler_params=pltpu.CompilerParams(
            dimension_semantics=("arbitrary",)),
    )(E)


def _sc_gather(table, idx_flat):
    nc, ns = 2, 16
    nw = nc * ns
    n_idx = idx_flat.shape[0]
    per_w = n_idx // nw
    mesh = plsc.VectorSubcoreMesh(core_axis_name="c", subcore_axis_name="s",
                                  num_cores=nc, num_subcores=ns)

    @functools.partial(
        pl.kernel,
        mesh=mesh,
        out_type=jax.ShapeDtypeStruct((n_idx, EMB_PAD), jnp.float32),
        scratch_types=[
            pltpu.VMEM((per_w,), jnp.int32),
            pltpu.VMEM((per_w, EMB_PAD), jnp.float32),
            pltpu.SemaphoreType.DMA,
        ],
    )
    def gather_kernel(table_hbm, idx_hbm, out_hbm, idx_v, rows_v, sem):
        wid = lax.axis_index("s") * nc + lax.axis_index("c")
        base = wid * per_w
        pltpu.sync_copy(idx_hbm.at[pl.ds(base, per_w)], idx_v)
        pltpu.async_copy(table_hbm.at[idx_v], rows_v, sem).wait()
        pltpu.sync_copy(rows_v, out_hbm.at[pl.ds(base, per_w)])

    return gather_kernel(table, idx_flat)


def _build_emb(rows_ref, emb_ref):
    emb_ref[:, 0:EMB_DIM] = rows_ref[0:BATCH_SIZE, 0:EMB_DIM]
    emb_ref[:, EMB_DIM:2 * EMB_DIM] = rows_ref[
        BATCH_SIZE:2 * BATCH_SIZE, 0:EMB_DIM]


def _pass1_body(rows_ref, w_ref, b_ref, m_ref, l_ref, emb_ref):
    j = pl.program_id(0)

    @pl.when(j == 0)
    def _():
        _build_emb(rows_ref, emb_ref)
        m_ref[...] = jnp.zeros_like(m_ref)
        l_ref[...] = jnp.zeros_like(l_ref)

    z = jnp.dot(emb_ref[...], w_ref[...],
                preferred_element_type=jnp.float32) + b_ref[...]
    r = jnp.maximum(z, 0.0)
    col = j * VT + lax.broadcasted_iota(jnp.int32, r.shape, 1)
    valid = col < VOCAB_SIZE
    r = jnp.where(valid, r, 0.0)
    tile_max = jnp.max(r, axis=1, keepdims=True)
    m_old = m_ref[...]
    m_new = jnp.maximum(m_old, tile_max)
    e = jnp.where(valid, jnp.exp(r - m_new), 0.0)
    l_ref[...] = l_ref[...] * jnp.exp(m_old - m_new) + jnp.sum(
        e, axis=1, keepdims=True)
    m_ref[...] = m_new


def _pass2_body(rows_ref, w_ref, b_ref, m_ref, l_ref, out_ref, emb_ref):
    j = pl.program_id(0)

    @pl.when(j == 0)
    def _():
        _build_emb(rows_ref, emb_ref)

    z = jnp.dot(emb_ref[...], w_ref[...],
                preferred_element_type=jnp.float32) + b_ref[...]
    r = jnp.maximum(z, 0.0)
    out_ref[...] = jnp.exp(r - m_ref[...]) * (1.0 / l_ref[...])


def kernel(inputs, E, W, b):
    table = _pad_table(E)
    idx = jnp.concatenate([inputs[:, 0], inputs[:, 1]]).astype(jnp.int32)
    rows = _sc_gather(table, idx)
    b2 = b.reshape(1, VOCAB_SIZE)

    m, l = pl.pallas_call(
        _pass1_body,
        grid=(NV,),
        in_specs=[
            pl.BlockSpec((2 * BATCH_SIZE, EMB_PAD), lambda j: (0, 0)),
            pl.BlockSpec((2 * EMB_DIM, VT), lambda j: (0, j)),
            pl.BlockSpec((1, VT), lambda j: (0, j)),
        ],
        out_specs=[
            pl.BlockSpec((BATCH_SIZE, 1), lambda j: (0, 0)),
            pl.BlockSpec((BATCH_SIZE, 1), lambda j: (0, 0)),
        ],
        out_shape=[
            jax.ShapeDtypeStruct((BATCH_SIZE, 1), jnp.float32),
            jax.ShapeDtypeStruct((BATCH_SIZE, 1), jnp.float32),
        ],
        scratch_shapes=[pltpu.VMEM((BATCH_SIZE, 2 * EMB_DIM), jnp.float32)],
        compiler_params=pltpu.CompilerParams(
            dimension_semantics=("arbitrary",)),
    )(rows, W, b2)

    out = pl.pallas_call(
        _pass2_body,
        grid=(NV,),
        in_specs=[
            pl.BlockSpec((2 * BATCH_SIZE, EMB_PAD), lambda j: (0, 0)),
            pl.BlockSpec((2 * EMB_DIM, VT), lambda j: (0, j)),
            pl.BlockSpec((1, VT), lambda j: (0, j)),
            pl.BlockSpec((BATCH_SIZE, 1), lambda j: (0, 0)),
            pl.BlockSpec((BATCH_SIZE, 1), lambda j: (0, 0)),
        ],
        out_specs=pl.BlockSpec((BATCH_SIZE, VT), lambda j: (0, j)),
        out_shape=jax.ShapeDtypeStruct((BATCH_SIZE, VOCAB_SIZE), jnp.float32),
        scratch_shapes=[pltpu.VMEM((BATCH_SIZE, 2 * EMB_DIM), jnp.float32)],
        compiler_params=pltpu.CompilerParams(
            dimension_semantics=("arbitrary",)),
    )(rows, W, b2, m, l)
    return out


# speedup vs baseline: 1.9797x; 1.5541x over previous
import functools

import jax
import jax.numpy as jnp
from jax import lax
from jax.experimental import pallas as pl
from jax.experimental.pallas import tpu as pltpu
from jax.experimental.pallas import tpu_sc as plsc

VOCAB_SIZE = 100000
EMB_DIM = 100
EMB_PAD = 128
BATCH_SIZE = 1024
VT = 2048
NV = (VOCAB_SIZE + VT - 1) // VT
PAD_ROWS = 10000


def _pad_body(e_ref, out_ref):
    out_ref[:, 0:EMB_DIM] = e_ref[...]
    out_ref[:, EMB_DIM:EMB_PAD] = jnp.zeros(
        (PAD_ROWS, EMB_PAD - EMB_DIM), jnp.float32)


def _pad_table(E):
    return pl.pallas_call(
        _pad_body,
        grid=(VOCAB_SIZE // PAD_ROWS,),
        in_specs=[pl.BlockSpec((PAD_ROWS, EMB_DIM), lambda i: (i, 0))],
        out_specs=pl.BlockSpec((PAD_ROWS, EMB_PAD), lambda i: (i, 0)),
        out_shape=jax.ShapeDtypeStruct((VOCAB_SIZE, EMB_PAD), jnp.float32),
        compiler_params=pltpu.CompilerParams(
            dimension_semantics=("arbitrary",)),
    )(E)


def _sc_gather(table, idx_flat):
    nc, ns = 2, 16
    nw = nc * ns
    n_idx = idx_flat.shape[0]
    per_w = n_idx // nw
    mesh = plsc.VectorSubcoreMesh(core_axis_name="c", subcore_axis_name="s",
                                  num_cores=nc, num_subcores=ns)

    @functools.partial(
        pl.kernel,
        mesh=mesh,
        out_type=jax.ShapeDtypeStruct((n_idx, EMB_PAD), jnp.float32),
        scratch_types=[
            pltpu.VMEM((per_w,), jnp.int32),
            pltpu.VMEM((per_w, EMB_PAD), jnp.float32),
            pltpu.SemaphoreType.DMA,
        ],
    )
    def gather_kernel(table_hbm, idx_hbm, out_hbm, idx_v, rows_v, sem):
        wid = lax.axis_index("s") * nc + lax.axis_index("c")
        base = wid * per_w
        pltpu.sync_copy(idx_hbm.at[pl.ds(base, per_w)], idx_v)
        pltpu.async_copy(table_hbm.at[idx_v], rows_v, sem).wait()
        pltpu.sync_copy(rows_v, out_hbm.at[pl.ds(base, per_w)])

    return gather_kernel(table, idx_flat)


def _build_embt(rows_t_ref, embt_ref):
    embt_ref[0:EMB_DIM, :] = rows_t_ref[0:EMB_DIM, 0:BATCH_SIZE]
    embt_ref[EMB_DIM:2 * EMB_DIM, :] = rows_t_ref[
        0:EMB_DIM, BATCH_SIZE:2 * BATCH_SIZE]


def _logits_t(w_ref, b_ref, embt_ref):
    zt = lax.dot_general(w_ref[...], embt_ref[...],
                         (((0,), (0,)), ((), ())),
                         preferred_element_type=jnp.float32)
    return zt + b_ref[...]


def _pass1_body(rows_t_ref, w_ref, b_ref, m_ref, l_ref, embt_ref):
    j = pl.program_id(0)

    @pl.when(j == 0)
    def _():
        _build_embt(rows_t_ref, embt_ref)
        m_ref[...] = jnp.zeros_like(m_ref)
        l_ref[...] = jnp.zeros_like(l_ref)

    r = jnp.maximum(_logits_t(w_ref, b_ref, embt_ref), 0.0)
    row = j * VT + lax.broadcasted_iota(jnp.int32, r.shape, 0)
    r = jnp.where(row < VOCAB_SIZE, r, -jnp.inf)
    tile_max = jnp.max(r, axis=0, keepdims=True)
    m_old = m_ref[...]
    m_new = jnp.maximum(m_old, tile_max)
    e = jnp.exp(r - m_new)
    l_ref[...] = l_ref[...] * jnp.exp(m_old - m_new) + jnp.sum(
        e, axis=0, keepdims=True)
    m_ref[...] = m_new


def _pass2_body(rows_t_ref, w_ref, b_ref, m_ref, l_ref, out_ref, embt_ref):
    j = pl.program_id(0)

    @pl.when(j == 0)
    def _():
        _build_embt(rows_t_ref, embt_ref)

    r = jnp.maximum(_logits_t(w_ref, b_ref, embt_ref), 0.0)
    out_ref[...] = jnp.exp(r - m_ref[...]) * (1.0 / l_ref[...])


def kernel(inputs, E, W, b):
    table = _pad_table(E)
    idx = jnp.concatenate([inputs[:, 0], inputs[:, 1]]).astype(jnp.int32)
    rows = _sc_gather(table, idx)
    rows_t = rows.T
    bc = b.reshape(VOCAB_SIZE, 1)

    m, l = pl.pallas_call(
        _pass1_body,
        grid=(NV,),
        in_specs=[
            pl.BlockSpec((EMB_PAD, 2 * BATCH_SIZE), lambda j: (0, 0)),
            pl.BlockSpec((2 * EMB_DIM, VT), lambda j: (0, j)),
            pl.BlockSpec((VT, 1), lambda j: (j, 0)),
        ],
        out_specs=[
            pl.BlockSpec((1, BATCH_SIZE), lambda j: (0, 0)),
            pl.BlockSpec((1, BATCH_SIZE), lambda j: (0, 0)),
        ],
        out_shape=[
            jax.ShapeDtypeStruct((1, BATCH_SIZE), jnp.float32),
            jax.ShapeDtypeStruct((1, BATCH_SIZE), jnp.float32),
        ],
        scratch_shapes=[pltpu.VMEM((2 * EMB_DIM, BATCH_SIZE), jnp.float32)],
        compiler_params=pltpu.CompilerParams(
            dimension_semantics=("arbitrary",)),
    )(rows_t, W, bc)

    out_t = pl.pallas_call(
        _pass2_body,
        grid=(NV,),
        in_specs=[
            pl.BlockSpec((EMB_PAD, 2 * BATCH_SIZE), lambda j: (0, 0)),
            pl.BlockSpec((2 * EMB_DIM, VT), lambda j: (0, j)),
            pl.BlockSpec((VT, 1), lambda j: (j, 0)),
            pl.BlockSpec((1, BATCH_SIZE), lambda j: (0, 0)),
            pl.BlockSpec((1, BATCH_SIZE), lambda j: (0, 0)),
        ],
        out_specs=pl.BlockSpec((VT, BATCH_SIZE), lambda j: (j, 0)),
        out_shape=jax.ShapeDtypeStruct((VOCAB_SIZE, BATCH_SIZE), jnp.float32),
        scratch_shapes=[pltpu.VMEM((2 * EMB_DIM, BATCH_SIZE), jnp.float32)],
        compiler_params=pltpu.CompilerParams(
            dimension_semantics=("arbitrary",)),
    )(rows_t, W, bc, m, l)
    return out_t.T
